# baseline (device time: 190990 ns/iter reference)
import functools

import jax
import jax.numpy as jnp
from jax import lax
from jax.experimental import pallas as pl
from jax.experimental.pallas import tpu as pltpu

N_DEV = 16
NH = 8
NSUB = 4
SH = NH // NSUB
SCALE = 64 ** -0.5
NEG_INF = -1e30


def _body(q_ref, k_ref, v_ref, out_ref, kvr_comm, kvl_comm,
          ot_acc, m_acc, l_acc,
          r_send_sems, r_recv_sems, l_send_sems, l_recv_sems):
    my = lax.axis_index("i")
    right = lax.rem(my + 1, N_DEV)
    left = lax.rem(my + N_DEV - 1, N_DEV)

    barrier_sem = pltpu.get_barrier_semaphore()
    for nbr in (left, right):
        pl.semaphore_signal(
            barrier_sem, inc=1, device_id=(nbr,),
            device_id_type=pl.DeviceIdType.MESH,
        )
    pl.semaphore_wait(barrier_sem, 2)

    for j in range(NSUB):
        kvr_comm[0, j, 0] = k_ref[j * SH:(j + 1) * SH]
        kvr_comm[0, j, 1] = v_ref[j * SH:(j + 1) * SH]
        kvl_comm[0, j, 0] = k_ref[NH + j * SH:NH + (j + 1) * SH]
        kvl_comm[0, j, 1] = v_ref[NH + j * SH:NH + (j + 1) * SH]

    chains = [
        (kvr_comm, r_send_sems, r_recv_sems, right, 0),
        (kvl_comm, l_send_sems, l_recv_sems, left, NH),
    ]

    def attn_step(rows, slot, j, comm, init=False):
        qt = q_ref[rows]
        kt = comm[slot, j, 0]
        vt = comm[slot, j, 1]
        s = lax.dot_general(
            qt, kt, (((1,), (1,)), ((0,), (0,))),
            preferred_element_type=jnp.float32,
        ) * SCALE
        if init:
            m_new = jnp.max(s, axis=-1)
        else:
            m_new = jnp.maximum(m_acc[rows], jnp.max(s, axis=-1))
            alpha = jnp.exp(m_acc[rows] - m_new)
        p = jnp.exp(s - m_new[:, :, None])
        pv = lax.dot_general(
            vt, p, (((2,), (2,)), ((0,), (0,))),
            preferred_element_type=jnp.float32,
        )
        if init:
            l_acc[rows] = jnp.sum(p, axis=-1)
            ot_acc[rows] = pv
        else:
            l_acc[rows] = l_acc[rows] * alpha + jnp.sum(p, axis=-1)
            ot_acc[rows] = ot_acc[rows] * alpha[:, None, :] + pv
        m_acc[rows] = m_new

    def compute_slot(slot, init=False):
        for comm, _, _, _, base in chains:
            for j in range(NSUB):
                attn_step(slice(base + j * SH, base + (j + 1) * SH),
                          slot, j, comm, init=init)

    def _rdma(comm, send_sems, recv_sems, h, j, dev):
        return pltpu.make_async_remote_copy(
            src_ref=comm.at[h, j],
            dst_ref=comm.at[h + 1, j],
            send_sem=send_sems.at[h, j],
            recv_sem=recv_sems.at[h, j],
            device_id=(dev,),
            device_id_type=pl.DeviceIdType.MESH,
        )

    def start_hop(h):
        for comm, ss, rs, dev, _ in chains:
            for j in range(NSUB):
                _rdma(comm, ss, rs, h, j, dev).start()

    start_hop(0)
    compute_slot(0, init=True)

    def hop(h, carry):
        for j in range(NSUB):
            for comm, ss, rs, dev, _base in chains:
                _rdma(comm, ss, rs, h, j, dev).wait_recv()
                _rdma(comm, ss, rs, h + 1, j, dev).start()
        compute_slot(h + 1)
        return carry

    lax.fori_loop(0, N_DEV - 2, hop, None)

    for comm, ss, rs, dev, _ in chains:
        for j in range(NSUB):
            _rdma(comm, ss, rs, N_DEV - 2, j, dev).wait_recv()
    compute_slot(N_DEV - 1)

    out_ref[...] = ot_acc[...] / l_acc[...][:, None, :]

    def drain(h, carry):
        for comm, ss, rs, dev, _base in chains:
            for j in range(NSUB):
                _rdma(comm, ss, rs, h, j, dev).wait_send()
        return carry

    lax.fori_loop(0, N_DEV - 1, drain, None)

    @functools.partial(pl.run_scoped, sem=pltpu.SemaphoreType.REGULAR)
    def _(sem):
        for nbr in (left, right):
            pl.semaphore_signal(
                sem, inc=1, device_id=(nbr,),
                device_id_type=pl.DeviceIdType.MESH,
            )
        pl.semaphore_wait(sem, 2)


def kernel(Q, K, V):
    b, s, h, d = Q.shape
    bh = b * h
    qt = Q.transpose(0, 2, 3, 1).reshape(bh, d, s)
    kt = K.transpose(0, 2, 3, 1).reshape(bh, d, s)
    vt = V.transpose(0, 2, 3, 1).reshape(bh, d, s)

    out = pl.pallas_call(
        _body,
        out_shape=jax.ShapeDtypeStruct((bh, d, s), jnp.float32),
        in_specs=[pl.BlockSpec(memory_space=pltpu.VMEM)] * 3,
        out_specs=pl.BlockSpec(memory_space=pltpu.VMEM),
        scratch_shapes=[
            pltpu.VMEM((N_DEV, NSUB, 2, SH, d, s), jnp.float32),
            pltpu.VMEM((N_DEV, NSUB, 2, SH, d, s), jnp.float32),
            pltpu.VMEM((bh, d, s), jnp.float32),
            pltpu.VMEM((bh, s), jnp.float32),
            pltpu.VMEM((bh, s), jnp.float32),
            pltpu.SemaphoreType.DMA((N_DEV - 1, NSUB)),
            pltpu.SemaphoreType.DMA((N_DEV - 1, NSUB)),
            pltpu.SemaphoreType.DMA((N_DEV - 1, NSUB)),
            pltpu.SemaphoreType.DMA((N_DEV - 1, NSUB)),
        ],
        compiler_params=pltpu.CompilerParams(
            collective_id=0, vmem_limit_bytes=60 * 1024 * 1024,
        ),
    )(qt, kt, vt)
    return out.reshape(b, h, d, s).transpose(0, 3, 1, 2)


# device time: 111456 ns/iter; 1.7136x vs baseline; 1.7136x over previous
import functools

import jax
import jax.numpy as jnp
from jax import lax
from jax.experimental import pallas as pl
from jax.experimental.pallas import tpu as pltpu

N_DEV = 16
NH = 8
NSUB = 2
SH = NH // NSUB
SCALE = 64 ** -0.5
NEG_INF = -1e30


def _body(q_ref, k_ref, v_ref, out_ref, kvr_comm, kvl_comm,
          ot_acc, m_acc, l_acc,
          r_send_sems, r_recv_sems, l_send_sems, l_recv_sems):
    my = lax.axis_index("i")
    right = lax.rem(my + 1, N_DEV)
    left = lax.rem(my + N_DEV - 1, N_DEV)

    barrier_sem = pltpu.get_barrier_semaphore()
    for nbr in (left, right):
        pl.semaphore_signal(
            barrier_sem, inc=1, device_id=(nbr,),
            device_id_type=pl.DeviceIdType.MESH,
        )
    pl.semaphore_wait(barrier_sem, 2)

    for j in range(NSUB):
        kvr_comm[0, j, 0] = k_ref[j * SH:(j + 1) * SH]
        kvr_comm[0, j, 1] = v_ref[j * SH:(j + 1) * SH]
        kvl_comm[0, j, 0] = k_ref[NH + j * SH:NH + (j + 1) * SH]
        kvl_comm[0, j, 1] = v_ref[NH + j * SH:NH + (j + 1) * SH]

    chains = [
        (kvr_comm, r_send_sems, r_recv_sems, right, 0),
        (kvl_comm, l_send_sems, l_recv_sems, left, NH),
    ]

    def attn_step(rows, slot, j, comm, init=False):
        qt = q_ref[rows]
        kt = comm[slot, j, 0]
        vt = comm[slot, j, 1]
        s = lax.dot_general(
            qt, kt, (((1,), (1,)), ((0,), (0,))),
            preferred_element_type=jnp.float32,
        ) * SCALE
        if init:
            m_new = jnp.max(s, axis=-1)
        else:
            m_new = jnp.maximum(m_acc[rows], jnp.max(s, axis=-1))
            alpha = jnp.exp(m_acc[rows] - m_new)
        p = jnp.exp(s - m_new[:, :, None])
        pv = lax.dot_general(
            vt, p.astype(jnp.bfloat16), (((2,), (2,)), ((0,), (0,))),
            preferred_element_type=jnp.float32,
        )
        if init:
            l_acc[rows] = jnp.sum(p, axis=-1)
            ot_acc[rows] = pv
        else:
            l_acc[rows] = l_acc[rows] * alpha + jnp.sum(p, axis=-1)
            ot_acc[rows] = ot_acc[rows] * alpha[:, None, :] + pv
        m_acc[rows] = m_new

    def compute_slot(slot, init=False):
        for comm, _, _, _, base in chains:
            for j in range(NSUB):
                attn_step(slice(base + j * SH, base + (j + 1) * SH),
                          slot, j, comm, init=init)

    def _rdma(comm, send_sems, recv_sems, h, j, dev):
        return pltpu.make_async_remote_copy(
            src_ref=comm.at[h, j],
            dst_ref=comm.at[h + 1, j],
            send_sem=send_sems.at[h, j],
            recv_sem=recv_sems.at[h, j],
            device_id=(dev,),
            device_id_type=pl.DeviceIdType.MESH,
        )

    def start_hop(h):
        for comm, ss, rs, dev, _ in chains:
            for j in range(NSUB):
                _rdma(comm, ss, rs, h, j, dev).start()

    start_hop(0)
    compute_slot(0, init=True)

    def hop(h, carry):
        for j in range(NSUB):
            for comm, ss, rs, dev, _base in chains:
                _rdma(comm, ss, rs, h, j, dev).wait_recv()
                _rdma(comm, ss, rs, h + 1, j, dev).start()
        compute_slot(h + 1)
        return carry

    lax.fori_loop(0, N_DEV - 2, hop, None)

    for comm, ss, rs, dev, _ in chains:
        for j in range(NSUB):
            _rdma(comm, ss, rs, N_DEV - 2, j, dev).wait_recv()
    compute_slot(N_DEV - 1)

    out_ref[...] = ot_acc[...] / l_acc[...][:, None, :]

    def drain(h, carry):
        for comm, ss, rs, dev, _base in chains:
            for j in range(NSUB):
                _rdma(comm, ss, rs, h, j, dev).wait_send()
        return carry

    lax.fori_loop(0, N_DEV - 1, drain, None)

    @functools.partial(pl.run_scoped, sem=pltpu.SemaphoreType.REGULAR)
    def _(sem):
        for nbr in (left, right):
            pl.semaphore_signal(
                sem, inc=1, device_id=(nbr,),
                device_id_type=pl.DeviceIdType.MESH,
            )
        pl.semaphore_wait(sem, 2)


def kernel(Q, K, V):
    b, s, h, d = Q.shape
    bh = b * h
    qt = Q.transpose(0, 2, 3, 1).reshape(bh, d, s).astype(jnp.bfloat16)
    kt = K.transpose(0, 2, 3, 1).reshape(bh, d, s).astype(jnp.bfloat16)
    vt = V.transpose(0, 2, 3, 1).reshape(bh, d, s).astype(jnp.bfloat16)

    out = pl.pallas_call(
        _body,
        out_shape=jax.ShapeDtypeStruct((bh, d, s), jnp.float32),
        in_specs=[pl.BlockSpec(memory_space=pltpu.VMEM)] * 3,
        out_specs=pl.BlockSpec(memory_space=pltpu.VMEM),
        scratch_shapes=[
            pltpu.VMEM((N_DEV, NSUB, 2, SH, d, s), jnp.bfloat16),
            pltpu.VMEM((N_DEV, NSUB, 2, SH, d, s), jnp.bfloat16),
            pltpu.VMEM((bh, d, s), jnp.float32),
            pltpu.VMEM((bh, s), jnp.float32),
            pltpu.VMEM((bh, s), jnp.float32),
            pltpu.SemaphoreType.DMA((N_DEV - 1, NSUB)),
            pltpu.SemaphoreType.DMA((N_DEV - 1, NSUB)),
            pltpu.SemaphoreType.DMA((N_DEV - 1, NSUB)),
            pltpu.SemaphoreType.DMA((N_DEV - 1, NSUB)),
        ],
        compiler_params=pltpu.CompilerParams(
            collective_id=0, vmem_limit_bytes=60 * 1024 * 1024,
        ),
    )(qt, kt, vt)
    return out.reshape(b, h, d, s).transpose(0, 3, 1, 2)
